# baseline (speedup 1.0000x reference)
import jax
import jax.numpy as jnp
from jax import lax
from jax.experimental import pallas as pl
from jax.experimental.pallas import tpu as pltpu
from jax.experimental.pallas import tpu_sc as plsc

NUM_CORES = 2
NUM_SUBCORES = 16
LANES = 16
NUM_WORKERS = NUM_CORES * NUM_SUBCORES

BATCH = 16384
EMBED_DIM = 64
SUPER = 2 * EMBED_DIM
NUM_ROWS = 1000000
NUM_SUPER = NUM_ROWS // 2
ROWS_PER_WORKER = BATCH // NUM_WORKERS
CHUNK = 128
NUM_CHUNKS = ROWS_PER_WORKER // CHUNK
GROUPS_PER_CHUNK = CHUNK // LANES

REPACK_BLOCK = 4000

_EXPONENT = -0.49
_LN2 = 0.6931471805599453
_GRAVITATION = 1e-07


def _repack_body(a_ref, b_ref, o_ref):
    o_ref[:, :EMBED_DIM] = a_ref[...]
    o_ref[:, EMBED_DIM:] = b_ref[...]


def _repack(table):
    nsteps = NUM_SUPER // REPACK_BLOCK
    return pl.pallas_call(
        _repack_body,
        grid=(nsteps,),
        in_specs=[
            pl.BlockSpec((REPACK_BLOCK, EMBED_DIM), lambda i: (i, 0)),
            pl.BlockSpec((REPACK_BLOCK, EMBED_DIM),
                         lambda i: (i + NUM_SUPER // REPACK_BLOCK, 0)),
        ],
        out_specs=pl.BlockSpec((REPACK_BLOCK, SUPER), lambda i: (i, 0)),
        out_shape=jax.ShapeDtypeStruct((NUM_SUPER, SUPER), jnp.float32),
    )(table, table)


def _sc_body(qtab, ctab, qsup, csup, qcol, ccol, out,
             qsup_v, csup_v, qcol_v, ccol_v,
             qbuf0, qbuf1, cbuf0, cbuf1, outbuf, sem):
    wid = lax.axis_index("s") * NUM_CORES + lax.axis_index("c")

    pltpu.sync_copy(qsup.at[wid], qsup_v)
    pltpu.sync_copy(csup.at[wid], csup_v)
    pltpu.sync_copy(qcol.at[wid], qcol_v)
    pltpu.sync_copy(ccol.at[wid], ccol_v)

    qbufs = [qbuf0, qbuf1]
    cbufs = [cbuf0, cbuf1]

    def issue(j):
        hq = pltpu.async_copy(qtab.at[qsup_v.at[j]], qbufs[j % 2], sem)
        hc = pltpu.async_copy(ctab.at[csup_v.at[j]], cbufs[j % 2], sem)
        return (hq, hc)

    lane = lax.iota(jnp.int32, LANES)
    zeros = jnp.zeros((LANES,), jnp.float32)

    handles = [issue(0)]
    cacc = zeros
    gacc = zeros
    for j in range(NUM_CHUNKS):
        hq, hc = handles[j]
        hq.wait()
        hc.wait()
        if j + 1 < NUM_CHUNKS:
            handles.append(issue(j + 1))
        qb = qbufs[j % 2]
        cb = cbufs[j % 2]

        def chunk_body(k, carry, j=j, qb=qb, cb=cb):
            cacc, gacc = carry
            rowv = k * LANES + lane
            jv = jnp.full((LANES,), j, jnp.int32)
            qc = plsc.load_gather(qcol_v, [jv, rowv])
            cc = plsc.load_gather(ccol_v, [jv, rowv])

            def dim_body(d, c3):
                dot, qn, cn = c3
                qv = plsc.load_gather(qb, [rowv, qc + d])
                cv = plsc.load_gather(cb, [rowv, cc + d])
                return dot + qv * cv, qn + qv * qv, cn + cv * cv

            dot, qn, cn = lax.fori_loop(
                0, EMBED_DIM, dim_body, (zeros, zeros, zeros), unroll=8)

            prod = qn * cn
            bits = plsc.bitcast(prod, jnp.int32)
            e = (bits >> 23) - 127
            mbits = (bits & 0x007FFFFF) | 0x3F800000
            m = plsc.bitcast(mbits, jnp.float32)
            t = (m - 1.0) / (m + 1.0)
            t2 = t * t
            poly = ((((t2 / 9.0 + 1.0 / 7.0) * t2 + 0.2) * t2 + 1.0 / 3.0)
                    * t2 + 1.0)
            ln_prod = e.astype(jnp.float32) * _LN2 + 2.0 * t * poly
            pw = jnp.exp(_EXPONENT * ln_prod)

            return cacc + dot * pw, gacc + (qn + cn)

        cacc, gacc = lax.fori_loop(
            0, GROUPS_PER_CHUNK, chunk_body, (cacc, gacc))

    outbuf[0, :] = cacc
    outbuf[1, :] = gacc
    pltpu.sync_copy(outbuf, out.at[wid])


@jax.jit
def _run(query_table, candidate_table, qsup_r, csup_r, qcol_r, ccol_r):
    qtab2 = _repack(query_table)
    ctab2 = _repack(candidate_table)
    mesh = plsc.VectorSubcoreMesh(
        core_axis_name="c", subcore_axis_name="s",
        num_cores=NUM_CORES, num_subcores=NUM_SUBCORES)
    parts = pl.kernel(
        _sc_body,
        out_type=jax.ShapeDtypeStruct((NUM_WORKERS, 2, LANES), jnp.float32),
        mesh=mesh,
        scratch_types=[
            pltpu.MemorySpace.VMEM((NUM_CHUNKS, CHUNK), jnp.int32),
            pltpu.MemorySpace.VMEM((NUM_CHUNKS, CHUNK), jnp.int32),
            pltpu.MemorySpace.VMEM((NUM_CHUNKS, CHUNK), jnp.int32),
            pltpu.MemorySpace.VMEM((NUM_CHUNKS, CHUNK), jnp.int32),
            pltpu.MemorySpace.VMEM((CHUNK, SUPER), jnp.float32),
            pltpu.MemorySpace.VMEM((CHUNK, SUPER), jnp.float32),
            pltpu.MemorySpace.VMEM((CHUNK, SUPER), jnp.float32),
            pltpu.MemorySpace.VMEM((CHUNK, SUPER), jnp.float32),
            pltpu.MemorySpace.VMEM((2, LANES), jnp.float32),
            pltpu.SemaphoreType.DMA,
        ],
        compiler_params=pltpu.CompilerParams(needs_layout_passes=False),
    )(qtab2, ctab2, qsup_r, csup_r, qcol_r, ccol_r)
    cos_loss = -jnp.sum(parts[:, 0, :])
    grav_loss = jnp.sum(parts[:, 1, :])
    return cos_loss + _GRAVITATION * grav_loss


def kernel(query_table, candidate_table, query_ids, candidate_ids):
    qids = query_ids.astype(jnp.int32)
    cids = candidate_ids.astype(jnp.int32)
    shape = (NUM_WORKERS, NUM_CHUNKS, CHUNK)
    qsup_r = (qids % NUM_SUPER).reshape(shape)
    csup_r = (cids % NUM_SUPER).reshape(shape)
    qcol_r = ((qids >= NUM_SUPER) * EMBED_DIM).reshape(shape)
    ccol_r = ((cids >= NUM_SUPER) * EMBED_DIM).reshape(shape)
    return _run(query_table, candidate_table, qsup_r, csup_r, qcol_r, ccol_r)


# speedup vs baseline: 1.3828x; 1.3828x over previous
import jax
import jax.numpy as jnp
from jax import lax
from jax.experimental import pallas as pl
from jax.experimental.pallas import tpu as pltpu
from jax.experimental.pallas import tpu_sc as plsc

NUM_CORES = 2
NUM_SUBCORES = 16
LANES = 16
NUM_WORKERS = NUM_CORES * NUM_SUBCORES

BATCH = 16384
EMBED_DIM = 64
PAIR = 2 * EMBED_DIM
ROWS_PER_WORKER = BATCH // NUM_WORKERS
CHUNK = 256
NUM_CHUNKS = ROWS_PER_WORKER // CHUNK
GROUPS_PER_CHUNK = CHUNK // LANES

GB = 512

_EXPONENT = -0.49
_LN2 = 0.6931471805599453
_GRAVITATION = 1e-07


def _tc_gather_body(qid_ref, cid_ref, qtab, ctab, o_ref, qbuf, cbuf, sem):
    def issue(r, _):
        pltpu.async_copy(
            qtab.at[pl.ds(qid_ref[r], 1)], qbuf.at[pl.ds(r, 1)], sem)
        pltpu.async_copy(
            ctab.at[pl.ds(cid_ref[r], 1)], cbuf.at[pl.ds(r, 1)], sem)
        return 0

    lax.fori_loop(0, GB, issue, 0, unroll=4)

    def drain(r, _):
        pltpu.make_async_copy(
            qtab.at[pl.ds(0, 1)], qbuf.at[pl.ds(0, 1)], sem).wait()
        pltpu.make_async_copy(
            ctab.at[pl.ds(0, 1)], cbuf.at[pl.ds(0, 1)], sem).wait()
        return 0

    lax.fori_loop(0, GB, drain, 0, unroll=4)

    o_ref[:, :EMBED_DIM] = qbuf[...]
    o_ref[:, EMBED_DIM:] = cbuf[...]


def _tc_gather(query_table, candidate_table, qids, cids):
    return pl.pallas_call(
        _tc_gather_body,
        grid=(BATCH // GB,),
        in_specs=[
            pl.BlockSpec((GB,), lambda i: (i,),
                         memory_space=pltpu.MemorySpace.SMEM),
            pl.BlockSpec((GB,), lambda i: (i,),
                         memory_space=pltpu.MemorySpace.SMEM),
            pl.BlockSpec(memory_space=pl.ANY),
            pl.BlockSpec(memory_space=pl.ANY),
        ],
        out_specs=pl.BlockSpec((GB, PAIR), lambda i: (i, 0)),
        out_shape=jax.ShapeDtypeStruct((BATCH, PAIR), jnp.float32),
        scratch_shapes=[
            pltpu.MemorySpace.VMEM((GB, EMBED_DIM), jnp.float32),
            pltpu.MemorySpace.VMEM((GB, EMBED_DIM), jnp.float32),
            pltpu.SemaphoreType.DMA,
        ],
    )(qids, cids, query_table, candidate_table)


def _sc_body(packed, out, buf0, buf1, outbuf, sem):
    wid = lax.axis_index("s") * NUM_CORES + lax.axis_index("c")
    base = wid * ROWS_PER_WORKER

    bufs = [buf0, buf1]
    handles = []
    for j in range(NUM_CHUNKS):
        handles.append(pltpu.async_copy(
            packed.at[pl.ds(base + j * CHUNK, CHUNK)], bufs[j], sem))

    lane = lax.iota(jnp.int32, LANES)
    zeros = jnp.zeros((LANES,), jnp.float32)

    cacc = zeros
    gacc = zeros
    for j in range(NUM_CHUNKS):
        handles[j].wait()
        buf = bufs[j]

        def chunk_body(k, carry, buf=buf):
            cacc, gacc = carry
            rowv = k * LANES + lane

            def dim_body(d, c3):
                dot, qn, cn = c3
                colv = jnp.full((LANES,), d, jnp.int32)
                qv = plsc.load_gather(buf, [rowv, colv])
                cv = plsc.load_gather(buf, [rowv, colv + EMBED_DIM])
                return dot + qv * cv, qn + qv * qv, cn + cv * cv

            dot, qn, cn = lax.fori_loop(
                0, EMBED_DIM, dim_body, (zeros, zeros, zeros), unroll=8)

            prod = qn * cn
            bits = plsc.bitcast(prod, jnp.int32)
            e = (bits >> 23) - 127
            mbits = (bits & 0x007FFFFF) | 0x3F800000
            m = plsc.bitcast(mbits, jnp.float32)
            t = (m - 1.0) / (m + 1.0)
            t2 = t * t
            poly = ((((t2 / 9.0 + 1.0 / 7.0) * t2 + 0.2) * t2 + 1.0 / 3.0)
                    * t2 + 1.0)
            ln_prod = e.astype(jnp.float32) * _LN2 + 2.0 * t * poly
            pw = jnp.exp(_EXPONENT * ln_prod)

            return cacc + dot * pw, gacc + (qn + cn)

        cacc, gacc = lax.fori_loop(
            0, GROUPS_PER_CHUNK, chunk_body, (cacc, gacc))

    outbuf[0, :] = cacc
    outbuf[1, :] = gacc
    pltpu.sync_copy(outbuf, out.at[wid])


@jax.jit
def _run(query_table, candidate_table, qids, cids):
    packed = _tc_gather(query_table, candidate_table, qids, cids)
    mesh = plsc.VectorSubcoreMesh(
        core_axis_name="c", subcore_axis_name="s",
        num_cores=NUM_CORES, num_subcores=NUM_SUBCORES)
    parts = pl.kernel(
        _sc_body,
        out_type=jax.ShapeDtypeStruct((NUM_WORKERS, 2, LANES), jnp.float32),
        mesh=mesh,
        scratch_types=[
            pltpu.MemorySpace.VMEM((CHUNK, PAIR), jnp.float32),
            pltpu.MemorySpace.VMEM((CHUNK, PAIR), jnp.float32),
            pltpu.MemorySpace.VMEM((2, LANES), jnp.float32),
            pltpu.SemaphoreType.DMA,
        ],
        compiler_params=pltpu.CompilerParams(needs_layout_passes=False),
    )(packed)
    cos_loss = -jnp.sum(parts[:, 0, :])
    grav_loss = jnp.sum(parts[:, 1, :])
    return cos_loss + _GRAVITATION * grav_loss


def kernel(query_table, candidate_table, query_ids, candidate_ids):
    qids = query_ids.astype(jnp.int32)
    cids = candidate_ids.astype(jnp.int32)
    return _run(query_table, candidate_table, qids, cids)


# speedup vs baseline: 1.3836x; 1.0006x over previous
import jax
import jax.numpy as jnp
from jax import lax
from jax.experimental import pallas as pl
from jax.experimental.pallas import tpu as pltpu
from jax.experimental.pallas import tpu_sc as plsc

NUM_CORES = 2
NUM_SUBCORES = 16
LANES = 16
NUM_WORKERS = NUM_CORES * NUM_SUBCORES

BATCH = 16384
EMBED_DIM = 64
PAIR = 2 * EMBED_DIM
ROWS_PER_WORKER = BATCH // NUM_WORKERS
CHUNK = 256
NUM_CHUNKS = ROWS_PER_WORKER // CHUNK
GROUPS_PER_CHUNK = CHUNK // LANES

GB = 512

_EXPONENT = -0.49
_LN2 = 0.6931471805599453
_GRAVITATION = 1e-07


def _tc_gather_body(qid_ref, cid_ref, qtab, ctab, o_ref, qbuf, cbuf, sem):
    def issue(r, _):
        pltpu.async_copy(
            qtab.at[pl.ds(qid_ref[r], 1)], qbuf.at[pl.ds(r, 1)], sem)
        pltpu.async_copy(
            ctab.at[pl.ds(cid_ref[r], 1)], cbuf.at[pl.ds(r, 1)], sem)
        return 0

    lax.fori_loop(0, GB, issue, 0, unroll=8)

    pltpu.make_async_copy(qtab.at[pl.ds(0, GB)], qbuf, sem).wait()
    pltpu.make_async_copy(ctab.at[pl.ds(0, GB)], cbuf, sem).wait()

    o_ref[:, :EMBED_DIM] = qbuf[...]
    o_ref[:, EMBED_DIM:] = cbuf[...]


def _tc_gather(query_table, candidate_table, qids, cids):
    return pl.pallas_call(
        _tc_gather_body,
        grid=(BATCH // GB,),
        in_specs=[
            pl.BlockSpec((GB,), lambda i: (i,),
                         memory_space=pltpu.MemorySpace.SMEM),
            pl.BlockSpec((GB,), lambda i: (i,),
                         memory_space=pltpu.MemorySpace.SMEM),
            pl.BlockSpec(memory_space=pl.ANY),
            pl.BlockSpec(memory_space=pl.ANY),
        ],
        out_specs=pl.BlockSpec((GB, PAIR), lambda i: (i, 0)),
        out_shape=jax.ShapeDtypeStruct((BATCH, PAIR), jnp.float32),
        scratch_shapes=[
            pltpu.MemorySpace.VMEM((GB, EMBED_DIM), jnp.float32),
            pltpu.MemorySpace.VMEM((GB, EMBED_DIM), jnp.float32),
            pltpu.SemaphoreType.DMA,
        ],
    )(qids, cids, query_table, candidate_table)


def _sc_body(packed, out, buf0, buf1, outbuf, sem):
    wid = lax.axis_index("s") * NUM_CORES + lax.axis_index("c")
    base = wid * ROWS_PER_WORKER

    bufs = [buf0, buf1]
    handles = []
    for j in range(NUM_CHUNKS):
        handles.append(pltpu.async_copy(
            packed.at[pl.ds(base + j * CHUNK, CHUNK)], bufs[j], sem))

    lane = lax.iota(jnp.int32, LANES)
    zeros = jnp.zeros((LANES,), jnp.float32)

    cacc = zeros
    gacc = zeros
    for j in range(NUM_CHUNKS):
        handles[j].wait()
        buf = bufs[j]

        def chunk_body(k, carry, buf=buf):
            cacc, gacc = carry
            rowv = k * LANES + lane

            def dim_body(d, c3):
                dot, qn, cn = c3
                colv = jnp.full((LANES,), d, jnp.int32)
                qv = plsc.load_gather(buf, [rowv, colv])
                cv = plsc.load_gather(buf, [rowv, colv + EMBED_DIM])
                return dot + qv * cv, qn + qv * qv, cn + cv * cv

            dot, qn, cn = lax.fori_loop(
                0, EMBED_DIM, dim_body, (zeros, zeros, zeros), unroll=8)

            prod = qn * cn
            bits = plsc.bitcast(prod, jnp.int32)
            e = (bits >> 23) - 127
            mbits = (bits & 0x007FFFFF) | 0x3F800000
            m = plsc.bitcast(mbits, jnp.float32)
            t = (m - 1.0) / (m + 1.0)
            t2 = t * t
            poly = ((((t2 / 9.0 + 1.0 / 7.0) * t2 + 0.2) * t2 + 1.0 / 3.0)
                    * t2 + 1.0)
            ln_prod = e.astype(jnp.float32) * _LN2 + 2.0 * t * poly
            pw = jnp.exp(_EXPONENT * ln_prod)

            return cacc + dot * pw, gacc + (qn + cn)

        cacc, gacc = lax.fori_loop(
            0, GROUPS_PER_CHUNK, chunk_body, (cacc, gacc))

    outbuf[0, :] = cacc
    outbuf[1, :] = gacc
    pltpu.sync_copy(outbuf, out.at[wid])


@jax.jit
def _run(query_table, candidate_table, qids, cids):
    packed = _tc_gather(query_table, candidate_table, qids, cids)
    mesh = plsc.VectorSubcoreMesh(
        core_axis_name="c", subcore_axis_name="s",
        num_cores=NUM_CORES, num_subcores=NUM_SUBCORES)
    parts = pl.kernel(
        _sc_body,
        out_type=jax.ShapeDtypeStruct((NUM_WORKERS, 2, LANES), jnp.float32),
        mesh=mesh,
        scratch_types=[
            pltpu.MemorySpace.VMEM((CHUNK, PAIR), jnp.float32),
            pltpu.MemorySpace.VMEM((CHUNK, PAIR), jnp.float32),
            pltpu.MemorySpace.VMEM((2, LANES), jnp.float32),
            pltpu.SemaphoreType.DMA,
        ],
        compiler_params=pltpu.CompilerParams(needs_layout_passes=False),
    )(packed)
    cos_loss = -jnp.sum(parts[:, 0, :])
    grav_loss = jnp.sum(parts[:, 1, :])
    return cos_loss + _GRAVITATION * grav_loss


def kernel(query_table, candidate_table, query_ids, candidate_ids):
    qids = query_ids.astype(jnp.int32)
    cids = candidate_ids.astype(jnp.int32)
    return _run(query_table, candidate_table, qids, cids)


# speedup vs baseline: 1.5688x; 1.1338x over previous
import jax
import jax.numpy as jnp
from jax import lax
from jax.experimental import pallas as pl
from jax.experimental.pallas import tpu as pltpu
from jax.experimental.pallas import tpu_sc as plsc

NUM_CORES = 2
NUM_SUBCORES = 16
LANES = 16
NUM_WORKERS = NUM_CORES * NUM_SUBCORES

BATCH = 16384
EMBED_DIM = 64
TC_ROWS = 8192
SC_ROWS = BATCH - TC_ROWS
ROWS_PER_WORKER = SC_ROWS // NUM_WORKERS
CHUNK = 128
NUM_CHUNKS = ROWS_PER_WORKER // CHUNK
GROUPS_PER_CHUNK = CHUNK // LANES

GB = 512

_EXPONENT = -0.49
_LN2 = 0.6931471805599453
_GRAVITATION = 1e-07


def _powm049(prod):
    bits = prod.view(jnp.int32) if hasattr(prod, "view") else prod
    bits = lax.bitcast_convert_type(prod, jnp.int32)
    e = (bits >> 23) - 127
    mbits = (bits & 0x007FFFFF) | 0x3F800000
    m = lax.bitcast_convert_type(mbits, jnp.float32)
    t = (m - 1.0) / (m + 1.0)
    t2 = t * t
    poly = ((((t2 / 9.0 + 1.0 / 7.0) * t2 + 0.2) * t2 + 1.0 / 3.0)
            * t2 + 1.0)
    ln_prod = e.astype(jnp.float32) * _LN2 + 2.0 * t * poly
    return jnp.exp(_EXPONENT * ln_prod)


def _tc_body(qid_ref, cid_ref, qtab, ctab, o_ref, qbuf, cbuf, sem):
    i = pl.program_id(0)

    def issue(r, _):
        pltpu.async_copy(
            qtab.at[pl.ds(qid_ref[r], 1)], qbuf.at[pl.ds(r, 1)], sem)
        pltpu.async_copy(
            ctab.at[pl.ds(cid_ref[r], 1)], cbuf.at[pl.ds(r, 1)], sem)
        return 0

    lax.fori_loop(0, GB, issue, 0, unroll=8)

    pltpu.make_async_copy(qtab.at[pl.ds(0, GB)], qbuf, sem).wait()
    pltpu.make_async_copy(ctab.at[pl.ds(0, GB)], cbuf, sem).wait()

    q = qbuf[...]
    c = cbuf[...]
    dot = jnp.sum(q * c, axis=1)
    qn = jnp.sum(q * q, axis=1)
    cn = jnp.sum(c * c, axis=1)
    pw = _powm049(qn * cn)
    cos_part = jnp.sum(dot * pw)
    grav_part = jnp.sum(qn + cn)

    @pl.when(i == 0)
    def _init():
        o_ref[...] = jnp.zeros_like(o_ref)

    o_ref[...] += jnp.concatenate(
        [jnp.full((1, 128), cos_part, jnp.float32),
         jnp.full((1, 128), grav_part, jnp.float32)], axis=0)


def _tc_loss(query_table, candidate_table, qids_a, cids_a):
    return pl.pallas_call(
        _tc_body,
        grid=(TC_ROWS // GB,),
        in_specs=[
            pl.BlockSpec((GB,), lambda i: (i,),
                         memory_space=pltpu.MemorySpace.SMEM),
            pl.BlockSpec((GB,), lambda i: (i,),
                         memory_space=pltpu.MemorySpace.SMEM),
            pl.BlockSpec(memory_space=pl.ANY),
            pl.BlockSpec(memory_space=pl.ANY),
        ],
        out_specs=pl.BlockSpec((2, 128), lambda i: (0, 0)),
        out_shape=jax.ShapeDtypeStruct((2, 128), jnp.float32),
        scratch_shapes=[
            pltpu.MemorySpace.VMEM((GB, EMBED_DIM), jnp.float32),
            pltpu.MemorySpace.VMEM((GB, EMBED_DIM), jnp.float32),
            pltpu.SemaphoreType.DMA,
        ],
    )(qids_a, cids_a, query_table, candidate_table)


def _sc_body(qtab, ctab, qids, cids, out,
             idx_stage, qbuf0, qbuf1, cbuf0, cbuf1, outbuf, sem):
    wid = lax.axis_index("s") * NUM_CORES + lax.axis_index("c")

    pltpu.sync_copy(qids.at[wid], idx_stage.at[0])
    pltpu.sync_copy(cids.at[wid], idx_stage.at[1])

    qbufs = [qbuf0, qbuf1]
    cbufs = [cbuf0, cbuf1]

    def issue_chunk(j):
        qb = qbufs[j % 2]
        cb = cbufs[j % 2]

        def issue_group(g, _):
            base = j * CHUNK + g * LANES
            qv = idx_stage[0, pl.ds(base, LANES)]
            cv = idx_stage[1, pl.ds(base, LANES)]
            for k in range(LANES):
                r = g * LANES + k
                pltpu.async_copy(
                    qtab.at[pl.ds(qv[k], 1)], qb.at[pl.ds(r, 1)], sem)
                pltpu.async_copy(
                    ctab.at[pl.ds(cv[k], 1)], cb.at[pl.ds(r, 1)], sem)
            return 0

        lax.fori_loop(0, GROUPS_PER_CHUNK, issue_group, 0)

    def drain_chunk(j):
        qb = qbufs[j % 2]
        cb = cbufs[j % 2]

        def drain_row(r, _):
            pltpu.make_async_copy(
                qtab.at[pl.ds(0, 1)], qb.at[pl.ds(0, 1)], sem).wait()
            pltpu.make_async_copy(
                ctab.at[pl.ds(0, 1)], cb.at[pl.ds(0, 1)], sem).wait()
            return 0

        lax.fori_loop(0, CHUNK, drain_row, 0)

    lane = lax.iota(jnp.int32, LANES)
    zeros = jnp.zeros((LANES,), jnp.float32)

    issue_chunk(0)
    cacc = zeros
    gacc = zeros
    for j in range(NUM_CHUNKS):
        drain_chunk(j)
        if j + 1 < NUM_CHUNKS:
            issue_chunk(j + 1)
        qb = qbufs[j % 2]
        cb = cbufs[j % 2]

        def chunk_body(k, carry, qb=qb, cb=cb):
            cacc, gacc = carry
            rowv = k * LANES + lane

            def dim_body(d, c3):
                dot, qn, cn = c3
                colv = jnp.full((LANES,), d, jnp.int32)
                qv = plsc.load_gather(qb, [rowv, colv])
                cv = plsc.load_gather(cb, [rowv, colv])
                return dot + qv * cv, qn + qv * qv, cn + cv * cv

            dot, qn, cn = lax.fori_loop(
                0, EMBED_DIM, dim_body, (zeros, zeros, zeros), unroll=8)

            prod = qn * cn
            bits = plsc.bitcast(prod, jnp.int32)
            e = (bits >> 23) - 127
            mbits = (bits & 0x007FFFFF) | 0x3F800000
            m = plsc.bitcast(mbits, jnp.float32)
            t = (m - 1.0) / (m + 1.0)
            t2 = t * t
            poly = ((((t2 / 9.0 + 1.0 / 7.0) * t2 + 0.2) * t2 + 1.0 / 3.0)
                    * t2 + 1.0)
            ln_prod = e.astype(jnp.float32) * _LN2 + 2.0 * t * poly
            pw = jnp.exp(_EXPONENT * ln_prod)

            return cacc + dot * pw, gacc + (qn + cn)

        cacc, gacc = lax.fori_loop(
            0, GROUPS_PER_CHUNK, chunk_body, (cacc, gacc))

    outbuf[0, :] = cacc
    outbuf[1, :] = gacc
    pltpu.sync_copy(outbuf, out.at[wid])


def _sc_loss(query_table, candidate_table, qids_b, cids_b):
    mesh = plsc.VectorSubcoreMesh(
        core_axis_name="c", subcore_axis_name="s",
        num_cores=NUM_CORES, num_subcores=NUM_SUBCORES)
    return pl.kernel(
        _sc_body,
        out_type=jax.ShapeDtypeStruct((NUM_WORKERS, 2, LANES), jnp.float32),
        mesh=mesh,
        scratch_types=[
            pltpu.MemorySpace.VMEM((2, ROWS_PER_WORKER), jnp.int32),
            pltpu.MemorySpace.VMEM((CHUNK, EMBED_DIM), jnp.float32),
            pltpu.MemorySpace.VMEM((CHUNK, EMBED_DIM), jnp.float32),
            pltpu.MemorySpace.VMEM((CHUNK, EMBED_DIM), jnp.float32),
            pltpu.MemorySpace.VMEM((CHUNK, EMBED_DIM), jnp.float32),
            pltpu.MemorySpace.VMEM((2, LANES), jnp.float32),
            pltpu.SemaphoreType.DMA,
        ],
        compiler_params=pltpu.CompilerParams(needs_layout_passes=False),
    )(query_table, candidate_table, qids_b, cids_b)


@jax.jit
def _run(query_table, candidate_table, qids, cids):
    qids_a, qids_b = qids[:TC_ROWS], qids[TC_ROWS:]
    cids_a, cids_b = cids[:TC_ROWS], cids[TC_ROWS:]
    sc_parts = _sc_loss(
        query_table, candidate_table,
        qids_b.reshape(NUM_WORKERS, ROWS_PER_WORKER),
        cids_b.reshape(NUM_WORKERS, ROWS_PER_WORKER))
    tc_parts = _tc_loss(query_table, candidate_table, qids_a, cids_a)
    cos_sum = tc_parts[0, 0] + jnp.sum(sc_parts[:, 0, :])
    grav_sum = tc_parts[1, 0] + jnp.sum(sc_parts[:, 1, :])
    return -cos_sum + _GRAVITATION * grav_sum


def kernel(query_table, candidate_table, query_ids, candidate_ids):
    qids = query_ids.astype(jnp.int32)
    cids = candidate_ids.astype(jnp.int32)
    return _run(query_table, candidate_table, qids, cids)


# speedup vs baseline: 1.6628x; 1.0599x over previous
import jax
import jax.numpy as jnp
from jax import lax
from jax.experimental import pallas as pl
from jax.experimental.pallas import tpu as pltpu
from jax.experimental.pallas import tpu_sc as plsc

NUM_CORES = 2
NUM_SUBCORES = 16
LANES = 16
NUM_WORKERS = NUM_CORES * NUM_SUBCORES

BATCH = 16384
EMBED_DIM = 64
ROWS_PER_WORKER = BATCH // NUM_WORKERS
CHUNK = 128
NUM_CHUNKS = ROWS_PER_WORKER // CHUNK
GROUPS_PER_CHUNK = CHUNK // LANES
NSEM = 8

_EXPONENT = -0.49
_LN2 = 0.6931471805599453
_GRAVITATION = 1e-07


def _sc_body(qtab, ctab, qids, cids, out,
             idx_stage,
             qbuf0, qbuf1, cbuf0, cbuf1, outbuf, *sems):
    wid = lax.axis_index("s") * NUM_CORES + lax.axis_index("c")

    pltpu.sync_copy(qids.at[wid], idx_stage.at[0])
    pltpu.sync_copy(cids.at[wid], idx_stage.at[1])

    qbufs = [qbuf0, qbuf1]
    cbufs = [cbuf0, cbuf1]

    def issue_chunk(j):
        qb = qbufs[j % 2]
        cb = cbufs[j % 2]

        def issue_group(g, _):
            base = j * CHUNK + g * LANES
            qv = idx_stage[0, pl.ds(base, LANES)]
            cv = idx_stage[1, pl.ds(base, LANES)]
            for k in range(LANES):
                r = g * LANES + k
                pltpu.async_copy(
                    qtab.at[pl.ds(qv[k], 1)],
                    qb.at[pl.ds(r, 1)], sems[k % NSEM])
                pltpu.async_copy(
                    ctab.at[pl.ds(cv[k], 1)],
                    cb.at[pl.ds(r, 1)], sems[(k + NSEM // 2) % NSEM])
            return 0

        lax.fori_loop(0, GROUPS_PER_CHUNK, issue_group, 0)

    def drain_chunk(j):
        qb = qbufs[j % 2]
        cb = cbufs[j % 2]

        def drain_row(r, _):
            for s in range(NSEM):
                pltpu.make_async_copy(
                    qtab.at[pl.ds(0, 1)],
                    qb.at[pl.ds(0, 1)], sems[s]).wait()
                pltpu.make_async_copy(
                    ctab.at[pl.ds(0, 1)],
                    cb.at[pl.ds(0, 1)], sems[s]).wait()
            return 0

        lax.fori_loop(0, CHUNK // NSEM, drain_row, 0)

    lane = lax.iota(jnp.int32, LANES)
    zeros = jnp.zeros((LANES,), jnp.float32)

    issue_chunk(0)
    cacc = zeros
    gacc = zeros
    for j in range(NUM_CHUNKS):
        drain_chunk(j)
        if j + 1 < NUM_CHUNKS:
            issue_chunk(j + 1)
        qb = qbufs[j % 2]
        cb = cbufs[j % 2]

        def chunk_body(k, carry, qb=qb, cb=cb):
            cacc, gacc = carry
            rowv = k * LANES + lane

            def dim_body(d, c3):
                dot, qn, cn = c3
                colv = jnp.full((LANES,), d, jnp.int32)
                qv = plsc.load_gather(qb, [rowv, colv])
                cv = plsc.load_gather(cb, [rowv, colv])
                return dot + qv * cv, qn + qv * qv, cn + cv * cv

            dot, qn, cn = lax.fori_loop(
                0, EMBED_DIM, dim_body, (zeros, zeros, zeros), unroll=8)

            prod = qn * cn
            bits = plsc.bitcast(prod, jnp.int32)
            e = (bits >> 23) - 127
            mbits = (bits & 0x007FFFFF) | 0x3F800000
            m = plsc.bitcast(mbits, jnp.float32)
            t = (m - 1.0) / (m + 1.0)
            t2 = t * t
            poly = ((((t2 / 9.0 + 1.0 / 7.0) * t2 + 0.2) * t2 + 1.0 / 3.0)
                    * t2 + 1.0)
            ln_prod = e.astype(jnp.float32) * _LN2 + 2.0 * t * poly
            pw = jnp.exp(_EXPONENT * ln_prod)

            return cacc + dot * pw, gacc + (qn + cn)

        cacc, gacc = lax.fori_loop(
            0, GROUPS_PER_CHUNK, chunk_body, (cacc, gacc))

    outbuf[0, :] = cacc
    outbuf[1, :] = gacc
    pltpu.sync_copy(outbuf, out.at[wid])


@jax.jit
def _run(query_table, candidate_table, qids_r, cids_r):
    mesh = plsc.VectorSubcoreMesh(
        core_axis_name="c", subcore_axis_name="s",
        num_cores=NUM_CORES, num_subcores=NUM_SUBCORES)
    parts = pl.kernel(
        _sc_body,
        out_type=jax.ShapeDtypeStruct((NUM_WORKERS, 2, LANES), jnp.float32),
        mesh=mesh,
        scratch_types=[
            pltpu.MemorySpace.VMEM((2, ROWS_PER_WORKER), jnp.int32),
            pltpu.MemorySpace.VMEM((CHUNK, EMBED_DIM), jnp.float32),
            pltpu.MemorySpace.VMEM((CHUNK, EMBED_DIM), jnp.float32),
            pltpu.MemorySpace.VMEM((CHUNK, EMBED_DIM), jnp.float32),
            pltpu.MemorySpace.VMEM((CHUNK, EMBED_DIM), jnp.float32),
            pltpu.MemorySpace.VMEM((2, LANES), jnp.float32),
        ] + [pltpu.SemaphoreType.DMA] * NSEM,
        compiler_params=pltpu.CompilerParams(needs_layout_passes=False),
    )(query_table, candidate_table, qids_r, cids_r)
    cos_loss = -jnp.sum(parts[:, 0, :])
    grav_loss = jnp.sum(parts[:, 1, :])
    return cos_loss + _GRAVITATION * grav_loss


def kernel(query_table, candidate_table, query_ids, candidate_ids):
    qids_r = query_ids.astype(jnp.int32).reshape(NUM_WORKERS, ROWS_PER_WORKER)
    cids_r = candidate_ids.astype(jnp.int32).reshape(
        NUM_WORKERS, ROWS_PER_WORKER)
    return _run(query_table, candidate_table, qids_r, cids_r)
